# baseline (speedup 1.0000x reference)
import functools

import jax
import jax.numpy as jnp
from jax import lax
from jax.experimental import pallas as pl
from jax.experimental.pallas import tpu as pltpu
from jax.experimental.pallas import tpu_sc as plsc

LREAL = 50
LMAIN = 48
LTAIL = LREAL - LMAIN
LANES = 16
NBUF = 4


def _accum_row(rows_v, zbuf_v, j):
  nd = rows_v.shape[1] // LANES

  def body(l, acc):
    return tuple(
        acc[d] + rows_v[l, pl.ds(d * LANES, LANES)] for d in range(nd)
    )

  init = tuple(rows_v[0, pl.ds(d * LANES, LANES)] for d in range(nd))
  acc = lax.fori_loop(1, LMAIN, body, init, unroll=4)
  for d in range(nd):
    zbuf_v[j, pl.ds(d * LANES, LANES)] = acc[d]


def _sc_embed(xmain, xtail, table):
  B = xmain.shape[0] // LMAIN
  D = table.shape[1]
  mesh = plsc.VectorSubcoreMesh(core_axis_name="c", subcore_axis_name="s")
  NW = mesh.num_cores * mesh.num_subcores
  bpw = B // NW
  steps = bpw // NBUF
  TROWS = NBUF * LTAIL

  @functools.partial(
      pl.kernel,
      out_type=jax.ShapeDtypeStruct((B, D), jnp.float32),
      mesh=mesh,
      scratch_types=[
          pltpu.VMEM((bpw * LMAIN,), jnp.int32),
          pltpu.VMEM((bpw * LTAIL,), jnp.int32),
          pltpu.VMEM((NBUF, LMAIN, D), jnp.float32),
          pltpu.VMEM((TROWS, D), jnp.float32),
          pltpu.VMEM((NBUF, D), jnp.float32),
          [pltpu.SemaphoreType.DMA] * NBUF,
          pltpu.SemaphoreType.DMA,
          pltpu.SemaphoreType.DMA,
      ],
  )
  def k(xmain_hbm, xtail_hbm, table_hbm, out_hbm, idx_v, tidx_v, bufs,
        tbuf, zbuf_v, sems, tsem, zsem):
    wid = lax.axis_index("s") * mesh.num_cores + lax.axis_index("c")
    base = wid * bpw

    def gather(r, b):
      pltpu.async_copy(
          table_hbm.at[idx_v.at[pl.ds(r * LMAIN, LMAIN)]], bufs.at[b],
          sems[b])

    def gather_wait(r, b):
      pltpu.make_async_copy(
          table_hbm.at[idx_v.at[pl.ds(r * LMAIN, LMAIN)]], bufs.at[b],
          sems[b]).wait()

    pltpu.sync_copy(xmain_hbm.at[pl.ds(base * LMAIN, bpw * LMAIN)], idx_v)
    pltpu.sync_copy(xtail_hbm.at[pl.ds(base * LTAIL, bpw * LTAIL)], tidx_v)

    def tail_gather(r0):
      return pltpu.async_copy(
          table_hbm.at[tidx_v.at[pl.ds(r0 * LTAIL, TROWS)]], tbuf, tsem)

    for b in range(NBUF):
      gather(b, b)
    tail_gather(0)

    def step(s, carry):
      r0 = NBUF * s
      @pl.when(s > 0)
      def _():
        pltpu.make_async_copy(
            zbuf_v, out_hbm.at[pl.ds(base, NBUF)], zsem).wait()

      for b in range(NBUF):
        gather_wait(r0 + b, b)
        _accum_row(bufs.at[b], zbuf_v, b)

        @pl.when(s < steps - 1)
        def _():
          gather(r0 + b + NBUF, b)

      pltpu.make_async_copy(
          table_hbm.at[tidx_v.at[pl.ds(r0 * LTAIL, TROWS)]], tbuf, tsem
      ).wait()
      for j in range(NBUF):
        for d in range(D // LANES):
          sl = pl.ds(d * LANES, LANES)
          zbuf_v[j, sl] = (
              zbuf_v[j, sl] + tbuf[LTAIL * j, sl] + tbuf[LTAIL * j + 1, sl])

      @pl.when(s < steps - 1)
      def _():
        tail_gather(r0 + NBUF)

      pltpu.async_copy(zbuf_v, out_hbm.at[pl.ds(base + r0, NBUF)], zsem)
      return carry

    lax.fori_loop(0, steps, step, 0)
    pltpu.make_async_copy(zbuf_v, out_hbm.at[pl.ds(base, NBUF)], zsem).wait()

  return k(xmain, xtail, table)


def _mlp_body(z_ref, w1_ref, b1_ref, w2_ref, b2_ref, w3_ref, b3_ref,
              turn_ref, o_ref):
  z = z_ref[...]
  h = lax.dot_general(z, w1_ref[...], (((1,), (1,)), ((), ())),
                      preferred_element_type=jnp.float32)
  h = jnp.clip(h + b1_ref[...], 0.0, 1.0)
  h = lax.dot_general(h, w2_ref[...], (((1,), (1,)), ((), ())),
                      preferred_element_type=jnp.float32)
  h = jnp.clip(h + b2_ref[...], 0.0, 1.0)
  o = jnp.sum(h * w3_ref[...], axis=1, keepdims=True) + b3_ref[...]
  o_ref[...] = o * turn_ref[...]


def _tc_mlp(z, W1, b1, W2, b2, W3, b3, turn):
  B, D = z.shape
  BT = 2048
  grid = B // BT
  return pl.pallas_call(
      _mlp_body,
      grid=(grid,),
      in_specs=[
          pl.BlockSpec((BT, D), lambda i: (i, 0)),
          pl.BlockSpec(W1.shape, lambda i: (0, 0)),
          pl.BlockSpec(b1.shape, lambda i: (0, 0)),
          pl.BlockSpec(W2.shape, lambda i: (0, 0)),
          pl.BlockSpec(b2.shape, lambda i: (0, 0)),
          pl.BlockSpec(W3.shape, lambda i: (0, 0)),
          pl.BlockSpec(b3.shape, lambda i: (0, 0)),
          pl.BlockSpec((BT, 1), lambda i: (i, 0)),
      ],
      out_specs=pl.BlockSpec((BT, 1), lambda i: (i, 0)),
      out_shape=jax.ShapeDtypeStruct((B, 1), jnp.float32),
  )(z, W1, b1, W2, b2, W3, b3, turn)


def kernel(x, turn, table, W1, b1, W2, b2, W3, b3):
  B, L = x.shape
  xi = x.astype(jnp.int32)
  outs = []
  H = B // 2
  for h in range(2):
    xh = xi[h * H:(h + 1) * H]
    z = _sc_embed(xh[:, :LMAIN].reshape(-1), xh[:, LMAIN:].reshape(-1),
                  table)
    outs.append(_tc_mlp(z, W1, b1.reshape(1, -1), W2, b2.reshape(1, -1),
                        W3, b3.reshape(1, 1), turn[h * H:(h + 1) * H]))
  return jnp.concatenate(outs, axis=0)


# speedup vs baseline: 1.0055x; 1.0055x over previous
import functools

import jax
import jax.numpy as jnp
from jax import lax
from jax.experimental import pallas as pl
from jax.experimental.pallas import tpu as pltpu
from jax.experimental.pallas import tpu_sc as plsc

LREAL = 50
LMAIN = 48
LTAIL = LREAL - LMAIN
LANES = 16
NBUF = 4


def _accum_row(rows_v, zbuf_v, j):
  nd = rows_v.shape[1] // LANES

  def body(l, acc):
    return tuple(
        acc[d] + rows_v[l, pl.ds(d * LANES, LANES)] for d in range(nd)
    )

  init = tuple(rows_v[0, pl.ds(d * LANES, LANES)] for d in range(nd))
  acc = lax.fori_loop(1, LMAIN, body, init, unroll=4)
  for d in range(nd):
    zbuf_v[j, pl.ds(d * LANES, LANES)] = acc[d]


def _sc_embed(xmain, xtail, table):
  B = xmain.shape[0] // LMAIN
  D = table.shape[1]
  mesh = plsc.VectorSubcoreMesh(core_axis_name="c", subcore_axis_name="s")
  NW = mesh.num_cores * mesh.num_subcores
  bpw = B // NW
  steps = bpw // NBUF
  TROWS = NBUF * LTAIL

  @functools.partial(
      pl.kernel,
      out_type=jax.ShapeDtypeStruct((B, D), jnp.float32),
      mesh=mesh,
      scratch_types=[
          pltpu.VMEM((bpw * LMAIN,), jnp.int32),
          pltpu.VMEM((bpw * LTAIL,), jnp.int32),
          pltpu.VMEM((NBUF, LMAIN, D), jnp.float32),
          pltpu.VMEM((TROWS, D), jnp.float32),
          pltpu.VMEM((NBUF, D), jnp.float32),
          [pltpu.SemaphoreType.DMA] * NBUF,
          pltpu.SemaphoreType.DMA,
          pltpu.SemaphoreType.DMA,
      ],
  )
  def k(xmain_hbm, xtail_hbm, table_hbm, out_hbm, idx_v, tidx_v, bufs,
        tbuf, zbuf_v, sems, tsem, zsem):
    wid = lax.axis_index("s") * mesh.num_cores + lax.axis_index("c")
    base = wid * bpw

    def gather(r, b):
      pltpu.async_copy(
          table_hbm.at[idx_v.at[pl.ds(r * LMAIN, LMAIN)]], bufs.at[b],
          sems[b])

    def gather_wait(r, b):
      pltpu.make_async_copy(
          table_hbm.at[idx_v.at[pl.ds(r * LMAIN, LMAIN)]], bufs.at[b],
          sems[b]).wait()

    pltpu.sync_copy(xmain_hbm.at[pl.ds(base * LMAIN, bpw * LMAIN)], idx_v)
    pltpu.sync_copy(xtail_hbm.at[pl.ds(base * LTAIL, bpw * LTAIL)], tidx_v)

    for b in range(NBUF):
      gather(b, b)

    def step(s, carry):
      r0 = NBUF * s
      tail_desc = pltpu.async_copy(
          table_hbm.at[tidx_v.at[pl.ds(r0 * LTAIL, TROWS)]], tbuf, tsem)

      @pl.when(s > 0)
      def _():
        pltpu.make_async_copy(
            zbuf_v, out_hbm.at[pl.ds(base, NBUF)], zsem).wait()

      for b in range(NBUF):
        gather_wait(r0 + b, b)
        _accum_row(bufs.at[b], zbuf_v, b)

        @pl.when(s < steps - 1)
        def _():
          gather(r0 + b + NBUF, b)

      tail_desc.wait()
      for j in range(NBUF):
        for d in range(D // LANES):
          sl = pl.ds(d * LANES, LANES)
          zbuf_v[j, sl] = (
              zbuf_v[j, sl] + tbuf[LTAIL * j, sl] + tbuf[LTAIL * j + 1, sl])

      pltpu.async_copy(zbuf_v, out_hbm.at[pl.ds(base + r0, NBUF)], zsem)
      return carry

    lax.fori_loop(0, steps, step, 0)
    pltpu.make_async_copy(zbuf_v, out_hbm.at[pl.ds(base, NBUF)], zsem).wait()

  return k(xmain, xtail, table)


def _mlp_body(z_ref, w1_ref, b1_ref, w2_ref, b2_ref, w3_ref, b3_ref,
              turn_ref, o_ref):
  z = z_ref[...]
  h = lax.dot_general(z, w1_ref[...], (((1,), (1,)), ((), ())),
                      preferred_element_type=jnp.float32)
  h = jnp.clip(h + b1_ref[...], 0.0, 1.0)
  h = lax.dot_general(h, w2_ref[...], (((1,), (1,)), ((), ())),
                      preferred_element_type=jnp.float32)
  h = jnp.clip(h + b2_ref[...], 0.0, 1.0)
  o = jnp.sum(h * w3_ref[...], axis=1, keepdims=True) + b3_ref[...]
  o_ref[...] = o * turn_ref[...]


def _tc_mlp(z, W1, b1, W2, b2, W3, b3, turn):
  B, D = z.shape
  BT = 2048
  grid = B // BT
  return pl.pallas_call(
      _mlp_body,
      grid=(grid,),
      in_specs=[
          pl.BlockSpec((BT, D), lambda i: (i, 0)),
          pl.BlockSpec(W1.shape, lambda i: (0, 0)),
          pl.BlockSpec(b1.shape, lambda i: (0, 0)),
          pl.BlockSpec(W2.shape, lambda i: (0, 0)),
          pl.BlockSpec(b2.shape, lambda i: (0, 0)),
          pl.BlockSpec(W3.shape, lambda i: (0, 0)),
          pl.BlockSpec(b3.shape, lambda i: (0, 0)),
          pl.BlockSpec((BT, 1), lambda i: (i, 0)),
      ],
      out_specs=pl.BlockSpec((BT, 1), lambda i: (i, 0)),
      out_shape=jax.ShapeDtypeStruct((B, 1), jnp.float32),
  )(z, W1, b1, W2, b2, W3, b3, turn)


def kernel(x, turn, table, W1, b1, W2, b2, W3, b3):
  B, L = x.shape
  xi = x.astype(jnp.int32)
  outs = []
  H = B // 2
  for h in range(2):
    xh = xi[h * H:(h + 1) * H]
    z = _sc_embed(xh[:, :LMAIN].reshape(-1), xh[:, LMAIN:].reshape(-1),
                  table)
    outs.append(_tc_mlp(z, W1, b1.reshape(1, -1), W2, b2.reshape(1, -1),
                        W3, b3.reshape(1, 1), turn[h * H:(h + 1) * H]))
  return jnp.concatenate(outs, axis=0)
